# baseline (speedup 1.0000x reference)
import functools
import math

import jax
import jax.numpy as jnp
from jax import lax
from jax.experimental import pallas as pl
from jax.experimental.pallas import tpu as pltpu
from jax.experimental.pallas import tpu_sc as plsc

N_NEURONS = 1024
SEQ_LEN = 2048
SIGMA = 2.0
N_EVENTS = 32768
N_BATCH = 16

NC, NS, LANES = 2, 16, 16
NW = NC * NS
ROWS = N_BATCH * N_NEURONS
WPR = N_NEURONS // 4
SLAB_ROWS = 128
SLAB_W = SLAB_ROWS * WPR
N_SLABS = ROWS // SLAB_ROWS
N_PASS = N_SLABS // NW
SLABS_PER_BATCH = N_NEURONS // SLAB_ROWS
ECH = 2048
SENTINEL_B = N_BATCH


def _sc_hist_body(time_hbm, neuron_hbm, batch_hbm, starts_hbm, counts_hbm,
                  slab_a, slab_b, sem_a, sem_b, sbuf, tbuf, nbuf, bbuf):
    slabs = (slab_a, slab_b)
    sems = (sem_a, sem_b)
    descs = [None, None]
    wid = lax.axis_index("s") * NC + lax.axis_index("c")

    pltpu.sync_copy(starts_hbm, sbuf)
    vec0 = sbuf[pl.ds(0, LANES)]
    vec1 = sbuf[pl.ds(LANES, LANES)]
    iota = lax.broadcasted_iota(jnp.int32, (LANES,), 0)

    def extract(i):
        lo = jnp.sum(jnp.where(iota == i, vec0, 0))
        hi = jnp.sum(jnp.where(iota == i - LANES, vec1, 0))
        return lo + hi

    one = jnp.full((LANES,), 1, jnp.int32)
    zeros = jnp.zeros((LANES,), jnp.int32)
    for p in range(N_PASS):
        s = p * NW + wid
        base_row = s * SLAB_ROWS
        b = s // SLABS_PER_BATCH
        start = extract(b)
        end = extract(b + 1)
        s0 = start & ~7
        n_ch = (end - s0 + (ECH - 1)) // ECH
        slab_v = slabs[p % 2]
        if descs[p % 2] is not None:
            descs[p % 2].wait()

        def zero_body(i, _, slab_v=slab_v):
            for u in range(8):
                slab_v[pl.ds((i * 8 + u) * LANES, LANES)] = zeros
            return 0

        lax.fori_loop(0, SLAB_W // (LANES * 8), zero_body, 0)

        def chunk_body(c, _, base_row=base_row, s0=s0, slab_v=slab_v):
            off = pl.multiple_of(s0 + c * ECH, 8)
            pltpu.sync_copy(time_hbm.at[pl.ds(off, ECH)], tbuf)
            pltpu.sync_copy(neuron_hbm.at[pl.ds(off, ECH)], nbuf)
            pltpu.sync_copy(batch_hbm.at[pl.ds(off, ECH)], bbuf)

            def scan_body(i, _):
                for u in range(4):
                    j = (i * 4 + u) * LANES
                    t = tbuf[pl.ds(j, LANES)]
                    n = nbuf[pl.ds(j, LANES)]
                    bb = bbuf[pl.ds(j, LANES)]
                    row = bb * N_NEURONS + n
                    lrow = row - base_row
                    m = (lrow >= 0) & (lrow < SLAB_ROWS)
                    word = lrow * WPR + (t & (WPR - 1))
                    val = one << ((t >> 8) << 3)
                    safe = jnp.where(m, word, 0)
                    plsc.addupdate_scatter(slab_v, [safe], val, mask=m)
                return 0

            lax.fori_loop(0, ECH // (LANES * 4), scan_body, 0)
            return 0

        lax.fori_loop(0, n_ch, chunk_body, 0)
        descs[p % 2] = pltpu.async_copy(
            slab_v, counts_hbm.at[pl.ds(s * SLAB_W, SLAB_W)], sems[p % 2])
    descs[0].wait()
    descs[1].wait()


@functools.lru_cache(maxsize=1)
def _sc_hist():
    return pl.kernel(
        _sc_hist_body,
        out_type=jax.ShapeDtypeStruct((N_SLABS * SLAB_W,), jnp.int32),
        mesh=plsc.VectorSubcoreMesh(
            core_axis_name="c", subcore_axis_name="s",
            num_cores=NC, num_subcores=NS),
        compiler_params=pltpu.CompilerParams(needs_layout_passes=False),
        scratch_types=[
            pltpu.VMEM((SLAB_W,), jnp.int32),
            pltpu.VMEM((SLAB_W,), jnp.int32),
            pltpu.SemaphoreType.DMA,
            pltpu.SemaphoreType.DMA,
            pltpu.VMEM((2 * LANES,), jnp.int32),
            pltpu.VMEM((ECH,), jnp.int32),
            pltpu.VMEM((ECH,), jnp.int32),
            pltpu.VMEM((ECH,), jnp.int32),
        ],
    )


TC_ROWS = 2048
PADL = 64
CPAD = 1280
NBLK = 9


def _tc_conv_body(cw_ref, out_ref, cpad_ref):
    cpad_ref[:, :PADL] = jnp.zeros((TC_ROWS, PADL), jnp.float32)
    w = cw_ref[...]
    for k in range(4):
        byte = (w >> (8 * k)) & 255
        cpad_ref[:, PADL + WPR * k:PADL + WPR * (k + 1)] = byte.astype(
            jnp.float32)
    cpad_ref[:, PADL + N_NEURONS:] = jnp.zeros(
        (TC_ROWS, CPAD - PADL - N_NEURONS), jnp.float32)
    x = lax.broadcasted_iota(jnp.int32, (256, 128), 0).astype(jnp.float32)
    tp = lax.broadcasted_iota(jnp.int32, (256, 128), 1).astype(jnp.float32)
    d = (tp + PADL - x) * (1.0 / SIGMA)
    g0 = jnp.exp(-0.5 * d * d) * (1.0 / (SIGMA * math.sqrt(2.0 * math.pi)))
    for j in range(NBLK):
        a = cpad_ref[:, 128 * j:128 * j + 256]
        out_ref[:, 128 * j:128 * j + 128] = jnp.dot(
            a, g0, preferred_element_type=jnp.float32)
    out_ref[:, NBLK * 128:] = jnp.zeros(
        (TC_ROWS, SEQ_LEN - NBLK * 128), jnp.float32)


def _tc_conv(counts_w):
    return pl.pallas_call(
        _tc_conv_body,
        grid=(ROWS // TC_ROWS,),
        in_specs=[pl.BlockSpec((TC_ROWS, WPR), lambda i: (i, 0))],
        out_specs=pl.BlockSpec((TC_ROWS, SEQ_LEN), lambda i: (i, 0)),
        out_shape=jax.ShapeDtypeStruct((ROWS, SEQ_LEN), jnp.float32),
        scratch_shapes=[pltpu.VMEM((TC_ROWS, CPAD), jnp.float32)],
    )(counts_w)


def kernel(events, batch_idx):
    time_i32 = events[:, 0].astype(jnp.int32)
    neuron_i32 = events[:, 1].astype(jnp.int32)
    batch_i32 = batch_idx.astype(jnp.int32)
    pad = jnp.zeros((ECH,), jnp.int32)
    time_p = jnp.concatenate([time_i32, pad])
    neuron_p = jnp.concatenate([neuron_i32, pad])
    batch_p = jnp.concatenate([batch_i32, pad + SENTINEL_B])
    starts = jnp.searchsorted(
        batch_i32, jnp.arange(N_BATCH + 1, dtype=jnp.int32)).astype(jnp.int32)
    starts_p = jnp.concatenate(
        [starts, jnp.zeros((2 * LANES - N_BATCH - 1,), jnp.int32)])
    counts_flat = _sc_hist()(time_p, neuron_p, batch_p, starts_p)
    counts_w = counts_flat.reshape(ROWS, WPR)
    out = _tc_conv(counts_w)
    return out.reshape(N_BATCH, N_NEURONS, SEQ_LEN)


# speedup vs baseline: 1.0541x; 1.0541x over previous
import functools
import math

import jax
import jax.numpy as jnp
from jax import lax
from jax.experimental import pallas as pl
from jax.experimental.pallas import tpu as pltpu
from jax.experimental.pallas import tpu_sc as plsc

N_NEURONS = 1024
SEQ_LEN = 2048
SIGMA = 2.0
N_EVENTS = 32768
N_BATCH = 16

NC, NS, LANES = 2, 16, 16
NW = NC * NS
ROWS = N_BATCH * N_NEURONS
WPR = N_NEURONS // 4
SLAB_ROWS = 256
SLAB_W = SLAB_ROWS * WPR
N_SLABS = ROWS // SLAB_ROWS
N_PASS = N_SLABS // NW
SLABS_PER_BATCH = N_NEURONS // SLAB_ROWS
ECH = 2048
SENTINEL_B = N_BATCH


def _sc_hist_body(time_hbm, neuron_hbm, batch_hbm, starts_hbm, counts_hbm,
                  slab_v, sbuf, tbuf, nbuf, bbuf):
    wid = lax.axis_index("s") * NC + lax.axis_index("c")

    pltpu.sync_copy(starts_hbm, sbuf)
    vec0 = sbuf[pl.ds(0, LANES)]
    vec1 = sbuf[pl.ds(LANES, LANES)]
    iota = lax.broadcasted_iota(jnp.int32, (LANES,), 0)

    def extract(i):
        lo = jnp.sum(jnp.where(iota == i, vec0, 0))
        hi = jnp.sum(jnp.where(iota == i - LANES, vec1, 0))
        return lo + hi

    one = jnp.full((LANES,), 1, jnp.int32)
    zeros = jnp.zeros((LANES,), jnp.int32)
    for p in range(N_PASS):
        s = p * NW + wid
        base_row = s * SLAB_ROWS
        b = s // SLABS_PER_BATCH
        start = extract(b)
        end = extract(b + 1)
        s0 = start & ~7
        n_ch = (end - s0 + (ECH - 1)) // ECH

        def zero_body(i, _):
            for u in range(8):
                slab_v[pl.ds((i * 8 + u) * LANES, LANES)] = zeros
            return 0

        lax.fori_loop(0, SLAB_W // (LANES * 8), zero_body, 0)

        def chunk_body(c, _, base_row=base_row, s0=s0):
            off = pl.multiple_of(s0 + c * ECH, 8)
            pltpu.sync_copy(time_hbm.at[pl.ds(off, ECH)], tbuf)
            pltpu.sync_copy(neuron_hbm.at[pl.ds(off, ECH)], nbuf)
            pltpu.sync_copy(batch_hbm.at[pl.ds(off, ECH)], bbuf)

            def scan_body(i, _):
                for u in range(4):
                    j = (i * 4 + u) * LANES
                    t = tbuf[pl.ds(j, LANES)]
                    n = nbuf[pl.ds(j, LANES)]
                    bb = bbuf[pl.ds(j, LANES)]
                    row = bb * N_NEURONS + n
                    lrow = row - base_row
                    m = (lrow >= 0) & (lrow < SLAB_ROWS)
                    word = lrow * WPR + (t & (WPR - 1))
                    val = one << ((t >> 8) << 3)
                    safe = jnp.where(m, word, 0)
                    plsc.addupdate_scatter(slab_v, [safe], val, mask=m)
                return 0

            lax.fori_loop(0, ECH // (LANES * 4), scan_body, 0)
            return 0

        lax.fori_loop(0, n_ch, chunk_body, 0)
        pltpu.sync_copy(slab_v, counts_hbm.at[pl.ds(s * SLAB_W, SLAB_W)])


@functools.lru_cache(maxsize=1)
def _sc_hist():
    return pl.kernel(
        _sc_hist_body,
        out_type=jax.ShapeDtypeStruct((N_SLABS * SLAB_W,), jnp.int32),
        mesh=plsc.VectorSubcoreMesh(
            core_axis_name="c", subcore_axis_name="s",
            num_cores=NC, num_subcores=NS),
        compiler_params=pltpu.CompilerParams(needs_layout_passes=False),
        scratch_types=[
            pltpu.VMEM((SLAB_W,), jnp.int32),
            pltpu.VMEM((2 * LANES,), jnp.int32),
            pltpu.VMEM((ECH,), jnp.int32),
            pltpu.VMEM((ECH,), jnp.int32),
            pltpu.VMEM((ECH,), jnp.int32),
        ],
    )


TC_ROWS = 2048
PADL = 64
CPAD = 1280
NBLK = 9


def _tc_conv_body(cw_ref, out_ref, cpad_ref):
    cpad_ref[:, :PADL] = jnp.zeros((TC_ROWS, PADL), jnp.float32)
    w = cw_ref[...]
    for k in range(4):
        byte = (w >> (8 * k)) & 255
        cpad_ref[:, PADL + WPR * k:PADL + WPR * (k + 1)] = byte.astype(
            jnp.float32)
    cpad_ref[:, PADL + N_NEURONS:] = jnp.zeros(
        (TC_ROWS, CPAD - PADL - N_NEURONS), jnp.float32)
    x = lax.broadcasted_iota(jnp.int32, (256, 128), 0).astype(jnp.float32)
    tp = lax.broadcasted_iota(jnp.int32, (256, 128), 1).astype(jnp.float32)
    d = (tp + PADL - x) * (1.0 / SIGMA)
    g0 = jnp.exp(-0.5 * d * d) * (1.0 / (SIGMA * math.sqrt(2.0 * math.pi)))
    for j in range(NBLK):
        a = cpad_ref[:, 128 * j:128 * j + 256]
        out_ref[:, 128 * j:128 * j + 128] = jnp.dot(
            a, g0, preferred_element_type=jnp.float32)
    out_ref[:, NBLK * 128:] = jnp.zeros(
        (TC_ROWS, SEQ_LEN - NBLK * 128), jnp.float32)


def _tc_conv(counts_w):
    return pl.pallas_call(
        _tc_conv_body,
        grid=(ROWS // TC_ROWS,),
        in_specs=[pl.BlockSpec((TC_ROWS, WPR), lambda i: (i, 0))],
        out_specs=pl.BlockSpec((TC_ROWS, SEQ_LEN), lambda i: (i, 0)),
        out_shape=jax.ShapeDtypeStruct((ROWS, SEQ_LEN), jnp.float32),
        scratch_shapes=[pltpu.VMEM((TC_ROWS, CPAD), jnp.float32)],
    )(counts_w)


def kernel(events, batch_idx):
    time_i32 = events[:, 0].astype(jnp.int32)
    neuron_i32 = events[:, 1].astype(jnp.int32)
    batch_i32 = batch_idx.astype(jnp.int32)
    pad = jnp.zeros((ECH,), jnp.int32)
    time_p = jnp.concatenate([time_i32, pad])
    neuron_p = jnp.concatenate([neuron_i32, pad])
    batch_p = jnp.concatenate([batch_i32, pad + SENTINEL_B])
    starts = jnp.searchsorted(
        batch_i32, jnp.arange(N_BATCH + 1, dtype=jnp.int32)).astype(jnp.int32)
    starts_p = jnp.concatenate(
        [starts, jnp.zeros((2 * LANES - N_BATCH - 1,), jnp.int32)])
    counts_flat = _sc_hist()(time_p, neuron_p, batch_p, starts_p)
    counts_w = counts_flat.reshape(ROWS, WPR)
    out = _tc_conv(counts_w)
    return out.reshape(N_BATCH, N_NEURONS, SEQ_LEN)


# speedup vs baseline: 1.0878x; 1.0319x over previous
import functools
import math

import jax
import jax.numpy as jnp
from jax import lax
from jax.experimental import pallas as pl
from jax.experimental.pallas import tpu as pltpu
from jax.experimental.pallas import tpu_sc as plsc

N_NEURONS = 1024
SEQ_LEN = 2048
SIGMA = 2.0
N_EVENTS = 32768
N_BATCH = 16

NC, NS, LANES = 2, 16, 16
NW = NC * NS
ROWS = N_BATCH * N_NEURONS
WPR = N_NEURONS // 4
SLAB_ROWS = 256
SLAB_W = SLAB_ROWS * WPR
N_SLABS = ROWS // SLAB_ROWS
N_PASS = N_SLABS // NW
SLABS_PER_BATCH = N_NEURONS // SLAB_ROWS
ECH = 2048
SENTINEL_B = N_BATCH


def _sc_hist_body(half, time_hbm, neuron_hbm, batch_hbm, starts_hbm,
                  counts_hbm, slab_v, sbuf, tbuf, nbuf, bbuf):
    wid = lax.axis_index("s") * NC + lax.axis_index("c")

    pltpu.sync_copy(starts_hbm, sbuf)
    vec0 = sbuf[pl.ds(0, LANES)]
    vec1 = sbuf[pl.ds(LANES, LANES)]
    iota = lax.broadcasted_iota(jnp.int32, (LANES,), 0)

    def extract(i):
        lo = jnp.sum(jnp.where(iota == i, vec0, 0))
        hi = jnp.sum(jnp.where(iota == i - LANES, vec1, 0))
        return lo + hi

    one = jnp.full((LANES,), 1, jnp.int32)
    zeros = jnp.zeros((LANES,), jnp.int32)
    s = half * NW + wid
    base_row = s * SLAB_ROWS
    b = s // SLABS_PER_BATCH
    start = extract(b)
    end = extract(b + 1)
    s0 = start & ~7
    n_ch = (end - s0 + (ECH - 1)) // ECH

    def zero_body(i, _):
        for u in range(8):
            slab_v[pl.ds((i * 8 + u) * LANES, LANES)] = zeros
        return 0

    lax.fori_loop(0, SLAB_W // (LANES * 8), zero_body, 0)

    def chunk_body(c, _):
        off = pl.multiple_of(s0 + c * ECH, 8)
        pltpu.sync_copy(time_hbm.at[pl.ds(off, ECH)], tbuf)
        pltpu.sync_copy(neuron_hbm.at[pl.ds(off, ECH)], nbuf)
        pltpu.sync_copy(batch_hbm.at[pl.ds(off, ECH)], bbuf)

        def scan_body(i, _):
            for u in range(4):
                j = (i * 4 + u) * LANES
                t = tbuf[pl.ds(j, LANES)]
                n = nbuf[pl.ds(j, LANES)]
                bb = bbuf[pl.ds(j, LANES)]
                row = bb * N_NEURONS + n
                lrow = row - base_row
                m = (lrow >= 0) & (lrow < SLAB_ROWS)
                word = lrow * WPR + (t & (WPR - 1))
                val = one << ((t >> 8) << 3)
                safe = jnp.where(m, word, 0)
                plsc.addupdate_scatter(slab_v, [safe], val, mask=m)
            return 0

        lax.fori_loop(0, ECH // (LANES * 4), scan_body, 0)
        return 0

    lax.fori_loop(0, n_ch, chunk_body, 0)
    pltpu.sync_copy(slab_v, counts_hbm.at[pl.ds(wid * SLAB_W, SLAB_W)])


@functools.lru_cache(maxsize=2)
def _sc_hist(half):
    return pl.kernel(
        functools.partial(_sc_hist_body, half),
        out_type=jax.ShapeDtypeStruct((NW * SLAB_W,), jnp.int32),
        mesh=plsc.VectorSubcoreMesh(
            core_axis_name="c", subcore_axis_name="s",
            num_cores=NC, num_subcores=NS),
        compiler_params=pltpu.CompilerParams(needs_layout_passes=False),
        scratch_types=[
            pltpu.VMEM((SLAB_W,), jnp.int32),
            pltpu.VMEM((2 * LANES,), jnp.int32),
            pltpu.VMEM((ECH,), jnp.int32),
            pltpu.VMEM((ECH,), jnp.int32),
            pltpu.VMEM((ECH,), jnp.int32),
        ],
    )


TC_ROWS = 2048
PADL = 64
CPAD = 1280
NBLK = 9


def _tc_conv_body(cw_ref, out_ref, cpad_ref):
    cpad_ref[:, :PADL] = jnp.zeros((TC_ROWS, PADL), jnp.float32)
    w = cw_ref[...]
    for k in range(4):
        byte = (w >> (8 * k)) & 255
        cpad_ref[:, PADL + WPR * k:PADL + WPR * (k + 1)] = byte.astype(
            jnp.float32)
    cpad_ref[:, PADL + N_NEURONS:] = jnp.zeros(
        (TC_ROWS, CPAD - PADL - N_NEURONS), jnp.float32)
    x = lax.broadcasted_iota(jnp.int32, (256, 128), 0).astype(jnp.float32)
    tp = lax.broadcasted_iota(jnp.int32, (256, 128), 1).astype(jnp.float32)
    d = (tp + PADL - x) * (1.0 / SIGMA)
    g0 = jnp.exp(-0.5 * d * d) * (1.0 / (SIGMA * math.sqrt(2.0 * math.pi)))
    for j in range(NBLK):
        a = cpad_ref[:, 128 * j:128 * j + 256]
        out_ref[:, 128 * j:128 * j + 128] = jnp.dot(
            a, g0, preferred_element_type=jnp.float32)
    out_ref[:, NBLK * 128:] = jnp.zeros(
        (TC_ROWS, SEQ_LEN - NBLK * 128), jnp.float32)


def _tc_conv_body_hi(cw_ref, prev_ref, out_ref, cpad_ref):
    del prev_ref
    _tc_conv_body(cw_ref, out_ref, cpad_ref)


HGRID = ROWS // 2 // TC_ROWS


def _tc_conv_lo(counts_w):
    return pl.pallas_call(
        _tc_conv_body,
        grid=(HGRID,),
        in_specs=[pl.BlockSpec((TC_ROWS, WPR), lambda i: (i, 0))],
        out_specs=pl.BlockSpec((TC_ROWS, SEQ_LEN), lambda i: (i, 0)),
        out_shape=jax.ShapeDtypeStruct((ROWS, SEQ_LEN), jnp.float32),
        scratch_shapes=[pltpu.VMEM((TC_ROWS, CPAD), jnp.float32)],
    )(counts_w)


def _tc_conv_hi(counts_w, prev):
    return pl.pallas_call(
        _tc_conv_body_hi,
        grid=(HGRID,),
        in_specs=[
            pl.BlockSpec((TC_ROWS, WPR), lambda i: (i, 0)),
            pl.BlockSpec(memory_space=pltpu.MemorySpace.HBM),
        ],
        out_specs=pl.BlockSpec((TC_ROWS, SEQ_LEN), lambda i: (i + HGRID, 0)),
        out_shape=jax.ShapeDtypeStruct((ROWS, SEQ_LEN), jnp.float32),
        scratch_shapes=[pltpu.VMEM((TC_ROWS, CPAD), jnp.float32)],
        input_output_aliases={1: 0},
    )(counts_w, prev)


def kernel(events, batch_idx):
    time_i32 = events[:, 0].astype(jnp.int32)
    neuron_i32 = events[:, 1].astype(jnp.int32)
    batch_i32 = batch_idx.astype(jnp.int32)
    pad = jnp.zeros((ECH,), jnp.int32)
    time_p = jnp.concatenate([time_i32, pad])
    neuron_p = jnp.concatenate([neuron_i32, pad])
    batch_p = jnp.concatenate([batch_i32, pad + SENTINEL_B])
    starts = jnp.searchsorted(
        batch_i32, jnp.arange(N_BATCH + 1, dtype=jnp.int32)).astype(jnp.int32)
    starts_p = jnp.concatenate(
        [starts, jnp.zeros((2 * LANES - N_BATCH - 1,), jnp.int32)])
    counts_lo = _sc_hist(0)(time_p, neuron_p, batch_p, starts_p)
    counts_hi = _sc_hist(1)(time_p, neuron_p, batch_p, starts_p)
    out_lo = _tc_conv_lo(counts_lo.reshape(ROWS // 2, WPR))
    out = _tc_conv_hi(counts_hi.reshape(ROWS // 2, WPR), out_lo)
    return out.reshape(N_BATCH, N_NEURONS, SEQ_LEN)
